# baseline (speedup 1.0000x reference)
import jax
import jax.numpy as jnp
from jax import lax
from jax.experimental import pallas as pl
from jax.experimental.pallas import tpu as pltpu
from jax.experimental.pallas import tpu_sc as plsc

NC = 2
NS = 16
NW = NC * NS
L = 16
B = 16384
D = 64
W = 128
Q = D // 2
BPW = B // NW
C = 128


def _pack_table(T):
    n = T.shape[0] * T.shape[1] // (2 * W)
    return jax.lax.bitcast_convert_type(
        T.astype(jnp.bfloat16).reshape(n, W, 2), jnp.int32)


def _sc_body(h_hbm, r_hbm, t_hbm, e1_hbm, e2_hbm, r1_hbm, r2_hbm, out_hbm,
             hidx_v, ridx_v, tidx_v, hph_v, rph_v, tph_v,
             e1h_v, e2h_v, r1_v, r2_v, e1t_v, e2t_v, out_v, sem):
    cid = lax.axis_index("c")
    sid = lax.axis_index("s")
    wid = sid * NC + cid
    base = wid * BPW
    lane = lax.iota(jnp.int32, L)

    def chunk(j, carry):
        off = base + j * C
        pltpu.sync_copy(h_hbm.at[pl.ds(off, C)], hidx_v)
        pltpu.sync_copy(r_hbm.at[pl.ds(off, C)], ridx_v)
        pltpu.sync_copy(t_hbm.at[pl.ds(off, C)], tidx_v)
        for q in range(C // L):
            sl = pl.ds(q * L, L)
            hph_v[sl] = lax.shift_right_logical(hidx_v[sl], 2)
            rph_v[sl] = lax.shift_right_logical(ridx_v[sl], 2)
            tph_v[sl] = lax.shift_right_logical(tidx_v[sl], 2)
        cps = [
            pltpu.async_copy(e1_hbm.at[hph_v], e1h_v, sem),
            pltpu.async_copy(e2_hbm.at[hph_v], e2h_v, sem),
            pltpu.async_copy(r1_hbm.at[rph_v], r1_v, sem),
            pltpu.async_copy(r2_hbm.at[rph_v], r2_v, sem),
            pltpu.async_copy(e1_hbm.at[tph_v], e1t_v, sem),
            pltpu.async_copy(e2_hbm.at[tph_v], e2t_v, sem),
        ]
        for cp in cps:
            cp.wait()

        def bf(v):
            return plsc.unpack(
                plsc.bitcast(v, jnp.bfloat16),
                format=plsc.PackFormat.INTERLEAVED)

        def group(g, carry2):
            gsl = pl.ds(g * L, L)
            hid16 = hidx_v[gsl]
            rid16 = ridx_v[gsl]
            tid16 = tidx_v[gsl]
            res = jnp.zeros((L,), jnp.float32)
            for k in range(L):
                i = g * L + k
                hoff = (hid16[k] & 3) * Q
                roff = (rid16[k] & 3) * Q
                toff = (tid16[k] & 3) * Q
                acc = jnp.zeros((L,), jnp.float32)
                for s in range(Q // L):
                    a0, a1 = bf(e1h_v[i, pl.ds(hoff + s * L, L)])
                    b0, b1 = bf(r1_v[i, pl.ds(roff + s * L, L)])
                    c0, c1 = bf(e2t_v[i, pl.ds(toff + s * L, L)])
                    d0, d1 = bf(e2h_v[i, pl.ds(hoff + s * L, L)])
                    e0, e1 = bf(r2_v[i, pl.ds(roff + s * L, L)])
                    f0, f1 = bf(e1t_v[i, pl.ds(toff + s * L, L)])
                    acc = (acc + a0 * b0 * c0 + a1 * b1 * c1
                           + d0 * e0 * f0 + d1 * e1 * f1)
                res = jnp.where(lane == k, jnp.sum(acc), res)
            out_v[pl.ds(g * L, L)] = res * 0.5
            return carry2

        lax.fori_loop(0, C // L, group, 0)
        pltpu.sync_copy(out_v, out_hbm.at[pl.ds(off, C)])
        return carry

    lax.fori_loop(0, BPW // C, chunk, 0)


def kernel(h_idx, r_idx, t_idx, E1, E2, R1, R2):
    h = h_idx.astype(jnp.int32)
    r = r_idx.astype(jnp.int32)
    t = t_idx.astype(jnp.int32)
    mesh = plsc.VectorSubcoreMesh(core_axis_name="c", subcore_axis_name="s")
    fn = pl.kernel(
        _sc_body,
        mesh=mesh,
        compiler_params=pltpu.CompilerParams(needs_layout_passes=False),
        out_type=jax.ShapeDtypeStruct((B,), jnp.float32),
        scratch_types=[
            pltpu.VMEM((C,), jnp.int32),
            pltpu.VMEM((C,), jnp.int32),
            pltpu.VMEM((C,), jnp.int32),
            pltpu.VMEM((C,), jnp.int32),
            pltpu.VMEM((C,), jnp.int32),
            pltpu.VMEM((C,), jnp.int32),
            pltpu.VMEM((C, W), jnp.int32),
            pltpu.VMEM((C, W), jnp.int32),
            pltpu.VMEM((C, W), jnp.int32),
            pltpu.VMEM((C, W), jnp.int32),
            pltpu.VMEM((C, W), jnp.int32),
            pltpu.VMEM((C, W), jnp.int32),
            pltpu.VMEM((C,), jnp.float32),
            pltpu.SemaphoreType.DMA,
        ],
    )
    return fn(h, r, t,
              _pack_table(E1), _pack_table(E2),
              _pack_table(R1), _pack_table(R2))


# speedup vs baseline: 51.7660x; 51.7660x over previous
import jax
import jax.numpy as jnp
from jax import lax
from jax.experimental import pallas as pl
from jax.experimental.pallas import tpu as pltpu
from jax.experimental.pallas import tpu_sc as plsc

NC = 2
NS = 16
NW = NC * NS
L = 16
B = 16384
D = 64
BPW = B // NW
C = 8
NCH = BPW // C
NSL = D // L


def _sc_body(h_hbm, r_hbm, t_hbm, e1_hbm, e2_hbm, r1_hbm, r2_hbm, out_hbm,
             hidx0, ridx0, tidx0, hidx1, ridx1, tidx1,
             e1h0, e2h0, r10, r20, e1t0, e2t0,
             e1h1, e2h1, r11, r21, e1t1, e2t1,
             out_v, sem0, sem1):
    cid = lax.axis_index("c")
    sid = lax.axis_index("s")
    wid = sid * NC + cid
    base = wid * BPW
    lane = lax.iota(jnp.int32, L)

    idx_bufs = ((hidx0, ridx0, tidx0), (hidx1, ridx1, tidx1))
    row_bufs = ((e1h0, e2h0, r10, r20, e1t0, e2t0),
                (e1h1, e2h1, r11, r21, e1t1, e2t1))
    sems = (sem0, sem1)

    def fire(jj, b):
        off = base + jj * C
        hb, rb, tb = idx_bufs[b]
        e1h, e2h, r1v, r2v, e1t, e2t = row_bufs[b]
        sem = sems[b]
        pltpu.sync_copy(h_hbm.at[pl.ds(off, C)], hb.at[pl.ds(0, C)])
        pltpu.sync_copy(r_hbm.at[pl.ds(off, C)], rb.at[pl.ds(0, C)])
        pltpu.sync_copy(t_hbm.at[pl.ds(off, C)], tb.at[pl.ds(0, C)])
        hid = hb[...]
        rid = rb[...]
        tid = tb[...]
        for k in range(C):
            hrow = pl.multiple_of((hid[k] >> 3) * 8, 8)
            rrow = pl.multiple_of((rid[k] >> 3) * 8, 8)
            trow = pl.multiple_of((tid[k] >> 3) * 8, 8)
            pltpu.async_copy(e1_hbm.at[pl.ds(hrow, 8), :], e1h.at[k], sem)
            pltpu.async_copy(e2_hbm.at[pl.ds(hrow, 8), :], e2h.at[k], sem)
            pltpu.async_copy(r1_hbm.at[pl.ds(rrow, 8), :], r1v.at[k], sem)
            pltpu.async_copy(r2_hbm.at[pl.ds(rrow, 8), :], r2v.at[k], sem)
            pltpu.async_copy(e1_hbm.at[pl.ds(trow, 8), :], e1t.at[k], sem)
            pltpu.async_copy(e2_hbm.at[pl.ds(trow, 8), :], e2t.at[k], sem)

    def drain_and_compute(jj, b, res):
        hb, rb, tb = idx_bufs[b]
        e1h, e2h, r1v, r2v, e1t, e2t = row_bufs[b]
        sem = sems[b]
        for buf in (e1h, e2h, r1v, r2v, e1t, e2t):
            pltpu.make_async_copy(
                e1_hbm.at[pl.ds(0, 8 * C), :], buf, sem).wait()
        hid = hb[...]
        rid = rb[...]
        tid = tb[...]
        for k in range(C):
            hs = hid[k] & 7
            rs = rid[k] & 7
            ts = tid[k] & 7
            acc = jnp.zeros((L,), jnp.float32)
            for s in range(NSL):
                sl = pl.ds(s * L, L)
                acc = (acc
                       + e1h[k, hs, sl] * r1v[k, rs, sl] * e2t[k, ts, sl]
                       + e2h[k, hs, sl] * r2v[k, rs, sl] * e1t[k, ts, sl])
            res = jnp.where(lane == b * C + k, jnp.sum(acc), res)
        return res

    fire(0, 0)

    def body(j2, carry):
        res = jnp.zeros((L,), jnp.float32)
        for b in (0, 1):
            jj = 2 * j2 + b

            @pl.when(jj + 1 < NCH)
            def _():
                fire(jj + 1, 1 - b)

            res = drain_and_compute(jj, b, res)
        out_v[...] = res * 0.5
        pltpu.sync_copy(out_v, out_hbm.at[pl.ds(base + j2 * L, L)])
        return carry

    lax.fori_loop(0, NCH // 2, body, 0)


def kernel(h_idx, r_idx, t_idx, E1, E2, R1, R2):
    h = h_idx.astype(jnp.int32)
    r = r_idx.astype(jnp.int32)
    t = t_idx.astype(jnp.int32)
    mesh = plsc.VectorSubcoreMesh(core_axis_name="c", subcore_axis_name="s")
    idx_t = pltpu.VMEM((L,), jnp.int32)
    row_t = pltpu.VMEM((C, 8, D), jnp.float32)
    fn = pl.kernel(
        _sc_body,
        mesh=mesh,
        compiler_params=pltpu.CompilerParams(needs_layout_passes=False),
        out_type=jax.ShapeDtypeStruct((B,), jnp.float32),
        scratch_types=(
            [idx_t] * 6 + [row_t] * 12
            + [pltpu.VMEM((L,), jnp.float32),
               pltpu.SemaphoreType.DMA, pltpu.SemaphoreType.DMA]
        ),
    )
    return fn(h, r, t, E1, E2, R1, R2)
